# baseline (speedup 1.0000x reference)
import functools

import jax
import jax.numpy as jnp
from jax import lax
from jax.experimental import pallas as pl
from jax.experimental.pallas import tpu as pltpu
from jax.experimental.pallas import tpu_sc as plsc

N = 10000
E = 320000
D_IN = 128
D_H1 = 128
D_H2 = 64

NC = 2
NS = 16
NW = NC * NS
CH = 128
NCHW = 80
HCH = 40
HF = 64
NBUF = 2 * (CH // HF)
EP = NW * NCHW * CH
ROWS2D = EP // CH
NPAD = 10240
PER_TILE = NPAD // NS
N_SINK = 64

_MESH = dict(core_axis_name="c", subcore_axis_name="s", num_cores=NC,
             num_subcores=NS)


@functools.partial(
    pl.kernel,
    out_type=jax.ShapeDtypeStruct((NC, NPAD), jnp.float32),
    mesh=plsc.VectorSubcoreMesh(**_MESH),
    scratch_types=[
        pltpu.VMEM((NCHW, CH), jnp.int32),
        pltpu.VMEM((CH,), jnp.float32),
        pltpu.VMEM_SHARED((NPAD,), jnp.float32),
    ],
)
def _sc_degree(ei_hbm, ones_hbm, zv_hbm, out_hbm, idx_v, ones_v, dacc):
    c = lax.axis_index("c")
    s = lax.axis_index("s")
    wid = s * NC + c
    base = s * PER_TILE
    pltpu.sync_copy(zv_hbm, dacc.at[pl.ds(base, PER_TILE)])
    pltpu.sync_copy(ei_hbm.at[1, pl.ds(wid * NCHW, NCHW)], idx_v)
    pltpu.sync_copy(ones_hbm, ones_v)
    plsc.subcore_barrier()

    def body(j, carry):
        pltpu.sync_copy(ones_v, dacc.at[idx_v.at[j]], add=True)
        return carry

    lax.fori_loop(0, NCHW, body, 0)
    plsc.subcore_barrier()
    pltpu.sync_copy(dacc.at[pl.ds(base, PER_TILE)],
                    out_hbm.at[c, pl.ds(base, PER_TILE)])


def _make_scatter(D):
    @functools.partial(
        pl.kernel,
        out_type=jax.ShapeDtypeStruct((NC, NPAD, D), jnp.float32),
        mesh=plsc.VectorSubcoreMesh(**_MESH),
        scratch_types=(
            [pltpu.VMEM((HCH, CH), jnp.int32),
             pltpu.VMEM((HCH, CH), jnp.int32)]
            + [pltpu.VMEM((HF, D), jnp.float32)] * NBUF
            + [pltpu.VMEM_SHARED((NPAD, D), jnp.float32)]
            + [pltpu.SemaphoreType.DMA] * (2 * NBUF)
        ),
    )
    def _scat(tab_hbm, ei_hbm, z_hbm, out_hbm, src_v, dst_v, *rest):
        rbufs = rest[:NBUF]
        acc = rest[NBUF]
        gsems = rest[NBUF + 1:2 * NBUF + 1]
        ssems = rest[2 * NBUF + 1:]
        c = lax.axis_index("c")
        s = lax.axis_index("s")
        wid = s * NC + c
        base = s * PER_TILE
        for k in range(PER_TILE // CH):
            pltpu.sync_copy(z_hbm, acc.at[pl.ds(base + k * CH, CH)])

        nsub = CH // HF
        qmax = HCH * nsub

        def sidx(v, q_static_i, b):
            return v.at[2 * q_static_i + b // nsub,
                        pl.ds((b % nsub) * HF, HF)]

        def gather(i, b):
            bb = b % NBUF
            pltpu.async_copy(tab_hbm.at[sidx(src_v, i, b)], rbufs[bb],
                             gsems[bb])

        def swait(i, b):
            bb = b % NBUF
            pltpu.make_async_copy(rbufs[bb], acc.at[sidx(dst_v, i, b)],
                                  ssems[bb]).wait()

        def body(i, carry):
            for b in range(NBUF):
                pltpu.make_async_copy(tab_hbm.at[sidx(src_v, i, b)],
                                      rbufs[b], gsems[b]).wait()
                pltpu.async_copy(rbufs[b], acc.at[sidx(dst_v, i, b)],
                                 ssems[b], add=True)
                q = NBUF * i + b
                if b < 2:
                    @pl.when(i > 0)
                    def _():
                        swait(i - 1, b + NBUF - 2)
                else:
                    swait(i, b - 2)

                @pl.when(q + 2 < qmax)
                def _():
                    if b + 2 < NBUF:
                        gather(i, b + 2)
                    else:
                        gather(i + 1, b + 2 - NBUF)
            return carry

        for h in range(NCHW // HCH):
            hb = wid * NCHW + h * HCH
            pltpu.sync_copy(ei_hbm.at[0, pl.ds(hb, HCH)], src_v)
            pltpu.sync_copy(ei_hbm.at[1, pl.ds(hb, HCH)], dst_v)
            gather(0, 0)
            gather(0, 1)
            if h == 0:
                plsc.subcore_barrier()
            lax.fori_loop(0, HCH // 2, body, 0)
            swait(HCH // 2 - 1, NBUF - 2)
            swait(HCH // 2 - 1, NBUF - 1)
        plsc.subcore_barrier()
        for k in range(PER_TILE // CH):
            sl = pl.ds(base + k * CH, CH)
            pltpu.sync_copy(acc.at[sl], out_hbm.at[c, sl])

    return _scat


_sc_scatter1 = _make_scatter(D_H1)


NB = 2048
_GRID = NPAD // NB

_full = lambda shp: pl.BlockSpec(shp, lambda i: (0,) * len(shp))


def _tc1_body(x_ref, w_ref, degp_ref, dinv_ref, hh_ref):
    i = pl.program_id(0)
    deg = (degp_ref[0, pl.ds(i * NB, NB)] + degp_ref[1, pl.ds(i * NB, NB)]
           + 1.0)
    dinv = lax.rsqrt(deg)[:, None]
    dinv_ref[...] = dinv
    h = jnp.dot(x_ref[...], w_ref[...], preferred_element_type=jnp.float32)
    hh_ref[...] = h * dinv


_tc1 = pl.pallas_call(
    _tc1_body,
    grid=(_GRID,),
    in_specs=[
        pl.BlockSpec((NB, D_IN), lambda i: (i, 0)),
        _full((D_IN, D_H1)),
        _full((NC, NPAD)),
    ],
    out_specs=(
        pl.BlockSpec((NB, 1), lambda i: (i, 0)),
        pl.BlockSpec((NB, D_H1), lambda i: (i, 0)),
    ),
    out_shape=(
        jax.ShapeDtypeStruct((N, 1), jnp.float32),
        jax.ShapeDtypeStruct((N, D_H1), jnp.float32),
    ),
)


def _tc2_body(acc_ref, hh1_ref, dinv_ref, b1_ref, w2_ref, hh2_ref):
    s = acc_ref[0] + acc_ref[1] + hh1_ref[...]
    o1 = jnp.maximum(dinv_ref[...] * s + b1_ref[...], 0.0)
    h2 = jnp.dot(o1, w2_ref[...], preferred_element_type=jnp.float32)
    hh2_ref[...] = h2 * dinv_ref[...]


_tc2 = pl.pallas_call(
    _tc2_body,
    grid=(_GRID,),
    in_specs=[
        pl.BlockSpec((NC, NB, D_H1), lambda i: (0, i, 0)),
        pl.BlockSpec((NB, D_H1), lambda i: (i, 0)),
        pl.BlockSpec((NB, 1), lambda i: (i, 0)),
        _full((1, D_H1)),
        _full((D_H1, D_H1)),
    ],
    out_specs=pl.BlockSpec((NB, D_H1), lambda i: (i, 0)),
    out_shape=jax.ShapeDtypeStruct((NPAD, D_H1), jnp.float32),
)


def _tc3_body(acc_ref, hh2_ref, dinv_ref, b2_ref, out_ref):
    out_ref[...] = (dinv_ref[...]
                    * (acc_ref[0, :, :D_H2] + acc_ref[1, :, :D_H2]
                       + hh2_ref[:, :D_H2])
                    + b2_ref[...])


_tc3 = pl.pallas_call(
    _tc3_body,
    grid=(_GRID,),
    in_specs=[
        pl.BlockSpec((NC, NB, D_H1), lambda i: (0, i, 0)),
        pl.BlockSpec((NB, D_H1), lambda i: (i, 0)),
        pl.BlockSpec((NB, 1), lambda i: (i, 0)),
        _full((1, D_H2)),
    ],
    out_specs=pl.BlockSpec((NB, D_H2), lambda i: (i, 0)),
    out_shape=jax.ShapeDtypeStruct((N, D_H2), jnp.float32),
)


def kernel(x, edge_index, W1, b1, W2, b2):
    pad = EP - E
    ar = jnp.arange(pad, dtype=jnp.int32)
    pad2 = jnp.stack([(ar * 131) % N, N + (ar % N_SINK)])
    ei = jnp.concatenate([edge_index.astype(jnp.int32), pad2],
                         axis=1).reshape(2, ROWS2D, CH)

    ones = jnp.ones((CH,), jnp.float32)
    zv = jnp.zeros((PER_TILE,), jnp.float32)
    z1 = jnp.zeros((CH, D_H1), jnp.float32)
    W2p = jnp.pad(W2, ((0, 0), (0, D_H1 - D_H2)))

    degp = _sc_degree(ei, ones, zv)

    dinv, hh1 = _tc1(x, W1, degp)

    acc1 = _sc_scatter1(hh1, ei, z1)
    hh2 = _tc2(acc1, hh1, dinv, b1[None, :], W2p)

    acc2 = _sc_scatter1(hh2, ei, z1)
    out = _tc3(acc2, hh2, dinv, b2[None, :])
    return out


# speedup vs baseline: 1.1892x; 1.1892x over previous
import functools

import jax
import jax.numpy as jnp
from jax import lax
from jax.experimental import pallas as pl
from jax.experimental.pallas import tpu as pltpu
from jax.experimental.pallas import tpu_sc as plsc

N = 10000
E = 320000
D_IN = 128
D_H1 = 128
D_H2 = 64

NC = 2
NS = 16
NW = NC * NS
CH = 128
NCHW = 80
HCH = 40
HF = 64
NBUF = 2 * (CH // HF)
EP = NW * NCHW * CH
ROWS2D = EP // CH
NPAD = 10240
PER_TILE = NPAD // NS
N_SINK = 64

_MESH = dict(core_axis_name="c", subcore_axis_name="s", num_cores=NC,
             num_subcores=NS)


@functools.partial(
    pl.kernel,
    out_type=jax.ShapeDtypeStruct((NC, NPAD), jnp.float32),
    mesh=plsc.VectorSubcoreMesh(**_MESH),
    scratch_types=[
        pltpu.VMEM((NCHW, CH), jnp.int32),
        pltpu.VMEM((CH,), jnp.float32),
        pltpu.VMEM_SHARED((NPAD,), jnp.float32),
    ],
)
def _sc_degree(ei_hbm, ones_hbm, zv_hbm, out_hbm, idx_v, ones_v, dacc):
    c = lax.axis_index("c")
    s = lax.axis_index("s")
    wid = s * NC + c
    base = s * PER_TILE
    pltpu.sync_copy(zv_hbm, dacc.at[pl.ds(base, PER_TILE)])
    pltpu.sync_copy(ei_hbm.at[1, pl.ds(wid * NCHW, NCHW)], idx_v)
    pltpu.sync_copy(ones_hbm, ones_v)
    plsc.subcore_barrier()

    def body(j, carry):
        pltpu.sync_copy(ones_v, dacc.at[idx_v.at[j]], add=True)
        return carry

    lax.fori_loop(0, NCHW, body, 0)
    plsc.subcore_barrier()
    pltpu.sync_copy(dacc.at[pl.ds(base, PER_TILE)],
                    out_hbm.at[c, pl.ds(base, PER_TILE)])


def _make_scatter(D):
    @functools.partial(
        pl.kernel,
        out_type=jax.ShapeDtypeStruct((NC, NPAD, D), jnp.float32),
        mesh=plsc.VectorSubcoreMesh(**_MESH),
        scratch_types=(
            [pltpu.VMEM((HCH, CH), jnp.int32),
             pltpu.VMEM((HCH, CH), jnp.int32)]
            + [pltpu.VMEM((HF, D), jnp.float32)] * NBUF
            + [pltpu.VMEM_SHARED((NPAD, D), jnp.float32)]
            + [pltpu.SemaphoreType.DMA] * NBUF
        ),
    )
    def _scat(tab_hbm, ei_hbm, z_hbm, out_hbm, src_v, dst_v, *rest):
        rbufs = rest[:NBUF]
        acc = rest[NBUF]
        gsems = rest[NBUF + 1:]
        c = lax.axis_index("c")
        s = lax.axis_index("s")
        wid = s * NC + c
        base = s * PER_TILE
        for k in range(PER_TILE // CH):
            pltpu.sync_copy(z_hbm, acc.at[pl.ds(base + k * CH, CH)])

        nsub = CH // HF
        qmax = HCH * nsub

        def sidx(v, q_static_i, b):
            return v.at[2 * q_static_i + b // nsub,
                        pl.ds((b % nsub) * HF, HF)]

        def body(i, carry):
            for b in range(NBUF):
                pltpu.make_async_copy(tab_hbm.at[sidx(src_v, i, b)],
                                      rbufs[b], gsems[b]).wait()
                pltpu.sync_copy(rbufs[b], acc.at[sidx(dst_v, i, b)],
                                add=True)

                @pl.when(i + 1 < HCH // 2)
                def _():
                    pltpu.async_copy(tab_hbm.at[sidx(src_v, i + 1, b)],
                                     rbufs[b], gsems[b])
            return carry

        for h in range(NCHW // HCH):
            hb = wid * NCHW + h * HCH
            pltpu.sync_copy(ei_hbm.at[0, pl.ds(hb, HCH)], src_v)
            pltpu.sync_copy(ei_hbm.at[1, pl.ds(hb, HCH)], dst_v)
            for b in range(NBUF):
                pltpu.async_copy(tab_hbm.at[sidx(src_v, 0, b)], rbufs[b],
                                 gsems[b])
            if h == 0:
                plsc.subcore_barrier()
            lax.fori_loop(0, HCH // 2, body, 0)
        plsc.subcore_barrier()
        for k in range(PER_TILE // CH):
            sl = pl.ds(base + k * CH, CH)
            pltpu.sync_copy(acc.at[sl], out_hbm.at[c, sl])

    return _scat


_sc_scatter1 = _make_scatter(D_H1)


NB = 2048
_GRID = NPAD // NB

_full = lambda shp: pl.BlockSpec(shp, lambda i: (0,) * len(shp))


def _tc1_body(x_ref, w_ref, degp_ref, dinv_ref, hh_ref):
    i = pl.program_id(0)
    deg = (degp_ref[0, pl.ds(i * NB, NB)] + degp_ref[1, pl.ds(i * NB, NB)]
           + 1.0)
    dinv = lax.rsqrt(deg)[:, None]
    dinv_ref[...] = dinv
    h = jnp.dot(x_ref[...], w_ref[...], preferred_element_type=jnp.float32)
    hh_ref[...] = h * dinv


_tc1 = pl.pallas_call(
    _tc1_body,
    grid=(_GRID,),
    in_specs=[
        pl.BlockSpec((NB, D_IN), lambda i: (i, 0)),
        _full((D_IN, D_H1)),
        _full((NC, NPAD)),
    ],
    out_specs=(
        pl.BlockSpec((NB, 1), lambda i: (i, 0)),
        pl.BlockSpec((NB, D_H1), lambda i: (i, 0)),
    ),
    out_shape=(
        jax.ShapeDtypeStruct((N, 1), jnp.float32),
        jax.ShapeDtypeStruct((N, D_H1), jnp.float32),
    ),
)


def _tc2_body(acc_ref, hh1_ref, dinv_ref, b1_ref, w2_ref, hh2_ref):
    s = acc_ref[0] + acc_ref[1] + hh1_ref[...]
    o1 = jnp.maximum(dinv_ref[...] * s + b1_ref[...], 0.0)
    h2 = jnp.dot(o1, w2_ref[...], preferred_element_type=jnp.float32)
    hh2_ref[...] = h2 * dinv_ref[...]


_tc2 = pl.pallas_call(
    _tc2_body,
    grid=(_GRID,),
    in_specs=[
        pl.BlockSpec((NC, NB, D_H1), lambda i: (0, i, 0)),
        pl.BlockSpec((NB, D_H1), lambda i: (i, 0)),
        pl.BlockSpec((NB, 1), lambda i: (i, 0)),
        _full((1, D_H1)),
        _full((D_H1, D_H1)),
    ],
    out_specs=pl.BlockSpec((NB, D_H1), lambda i: (i, 0)),
    out_shape=jax.ShapeDtypeStruct((NPAD, D_H1), jnp.float32),
)


def _tc3_body(acc_ref, hh2_ref, dinv_ref, b2_ref, out_ref):
    out_ref[...] = (dinv_ref[...]
                    * (acc_ref[0, :, :D_H2] + acc_ref[1, :, :D_H2]
                       + hh2_ref[:, :D_H2])
                    + b2_ref[...])


_tc3 = pl.pallas_call(
    _tc3_body,
    grid=(_GRID,),
    in_specs=[
        pl.BlockSpec((NC, NB, D_H1), lambda i: (0, i, 0)),
        pl.BlockSpec((NB, D_H1), lambda i: (i, 0)),
        pl.BlockSpec((NB, 1), lambda i: (i, 0)),
        _full((1, D_H2)),
    ],
    out_specs=pl.BlockSpec((NB, D_H2), lambda i: (i, 0)),
    out_shape=jax.ShapeDtypeStruct((N, D_H2), jnp.float32),
)


def kernel(x, edge_index, W1, b1, W2, b2):
    pad = EP - E
    ar = jnp.arange(pad, dtype=jnp.int32)
    pad2 = jnp.stack([(ar * 131) % N, N + (ar % N_SINK)])
    ei = jnp.concatenate([edge_index.astype(jnp.int32), pad2],
                         axis=1).reshape(2, ROWS2D, CH)

    ones = jnp.ones((CH,), jnp.float32)
    zv = jnp.zeros((PER_TILE,), jnp.float32)
    z1 = jnp.zeros((CH, D_H1), jnp.float32)
    W2p = jnp.pad(W2, ((0, 0), (0, D_H1 - D_H2)))

    degp = _sc_degree(ei, ones, zv)

    dinv, hh1 = _tc1(x, W1, degp)

    acc1 = _sc_scatter1(hh1, ei, z1)
    hh2 = _tc2(acc1, hh1, dinv, b1[None, :], W2p)

    acc2 = _sc_scatter1(hh2, ei, z1)
    out = _tc3(acc2, hh2, dinv, b2[None, :])
    return out
